# baseline (speedup 1.0000x reference)
import functools

import jax
import jax.numpy as jnp
from jax import lax
from jax.experimental import pallas as pl
from jax.experimental.pallas import tpu as pltpu
from jax.experimental.pallas import tpu_sc as plsc

_LAMBDA = 0.1
_SMOOTH = 1e-07


def _tc_body(B, C, S, num_h):
    def body(sup_ref, cs_ref, s2s_ref, tgt_ref, w_ref,
             loss_ref, fin_ref,
             a_interc, a_sumc, a_cntc, a_sums, a_cnts, a_inters):
        b = pl.program_id(0)
        h = pl.program_id(1)

        @pl.when(jnp.logical_and(b == 0, h == 0))
        def _init():
            for c in range(C):
                a_interc[c] = 0.0
                a_sumc[c] = 0.0
                a_cntc[c] = 0.0
            for s in range(S):
                a_sums[s] = 0.0
                a_cnts[s] = 0.0
                a_inters[s] = 0.0

        t = tgt_ref[0]
        st = jnp.zeros_like(t)
        for c in range(C):
            oh = t == c
            ohf = oh.astype(jnp.float32)
            x = cs_ref[0, c]
            sidx = s2s_ref[c]
            g = sup_ref[0, sidx]
            fin_ref[0, c] = x * g
            a_interc[c] += jnp.sum(x * ohf)
            a_sumc[c] += jnp.sum(x)
            a_cntc[c] += jnp.sum(ohf)
            st = st + sidx * oh.astype(jnp.int32)
        for s in range(S):
            sup_s = sup_ref[0, s]
            m = (st == s).astype(jnp.float32)
            a_sums[s] += jnp.sum(sup_s)
            a_cnts[s] += jnp.sum(m)
            a_inters[s] += jnp.sum(sup_s * m)

        @pl.when(jnp.logical_and(b == B - 1, h == num_h - 1))
        def _finish():
            sl = 0.0
            for s in range(S):
                sl += 1.0 - (2.0 * a_inters[s] + _SMOOTH) / (
                    a_sums[s] + a_cnts[s] + _SMOOTH)
            cl = 0.0
            wsum = 0.0
            for c in range(C):
                pc = 1.0 - (2.0 * a_interc[c] + _SMOOTH) / (
                    a_sumc[c] + a_cntc[c] + _SMOOTH)
                cl += pc * w_ref[c]
                wsum += w_ref[c]
            loss_ref[0, 0] = _LAMBDA * sl / S + cl / wsum

    return body


def _make_sc_onehot(B, C, HW):
    NW = 32
    WPB = NW // B
    span = HW // WPB
    CH = 2048
    n_chunks = span // CH
    L = 16

    @functools.partial(
        pl.kernel,
        out_type=jax.ShapeDtypeStruct((B * C * HW,), jnp.float32),
        mesh=plsc.VectorSubcoreMesh(core_axis_name="c", subcore_axis_name="s"),
        scratch_types=[
            pltpu.VMEM((CH,), jnp.int32),
            pltpu.VMEM((C, CH), jnp.float32),
        ],
    )
    def sc_onehot(tgt_hbm, out_hbm, t_v, oh_v):
        wid = lax.axis_index("s") * 2 + lax.axis_index("c")
        b = wid // WPB
        q = wid % WPB
        base_in = b * HW + q * span

        def chunk_body(k, _):
            off = base_in + k * CH
            pltpu.sync_copy(tgt_hbm.at[pl.ds(off, CH)], t_v)

            def vec_body(v, _):
                tv = t_v[pl.ds(v * L, L)]
                for c in range(C):
                    oh_v[c, pl.ds(v * L, L)] = jnp.where(
                        tv == c, 1.0, 0.0).astype(jnp.float32)
                return 0

            lax.fori_loop(0, CH // L, vec_body, 0)
            for c in range(C):
                out_off = (b * C + c) * HW + q * span + k * CH
                pltpu.sync_copy(oh_v.at[c], out_hbm.at[pl.ds(out_off, CH)])
            return 0

        lax.fori_loop(0, n_chunks, chunk_body, 0)

    return sc_onehot


def kernel(superclass_scores, class_score, super2sub, target, weights):
    B, C, H, W = class_score.shape
    S = superclass_scores.shape[1]
    HW = H * W
    bh = 128
    num_h = H // bh

    cids = jnp.arange(C, dtype=jnp.int32)
    member = jnp.any(super2sub.astype(jnp.int32)[None, :, :] == cids[:, None, None],
                     axis=2)
    sub2super = jnp.sum(member.astype(jnp.int32)
                        * jnp.arange(S, dtype=jnp.int32)[None, :], axis=1)

    oh_flat = _make_sc_onehot(B, C, HW)(target.reshape(-1))
    oh = oh_flat.reshape(B, C, H, W)

    grid = (B, num_h)
    out_shapes = (
        jax.ShapeDtypeStruct((1, 1), jnp.float32),
        jax.ShapeDtypeStruct((B, C, H, W), jnp.float32),
    )
    loss2d, fin = pl.pallas_call(
        _tc_body(B, C, S, num_h),
        grid=grid,
        in_specs=[
            pl.BlockSpec((1, S, bh, W), lambda b, h: (b, 0, h, 0)),
            pl.BlockSpec((1, C, bh, W), lambda b, h: (b, 0, h, 0)),
            pl.BlockSpec(memory_space=pltpu.SMEM),
            pl.BlockSpec((1, bh, W), lambda b, h: (b, h, 0)),
            pl.BlockSpec(memory_space=pltpu.SMEM),
        ],
        out_specs=(
            pl.BlockSpec(memory_space=pltpu.SMEM),
            pl.BlockSpec((1, C, bh, W), lambda b, h: (b, 0, h, 0)),
        ),
        scratch_shapes=[
            pltpu.SMEM((C,), jnp.float32),
            pltpu.SMEM((C,), jnp.float32),
            pltpu.SMEM((C,), jnp.float32),
            pltpu.SMEM((S,), jnp.float32),
            pltpu.SMEM((S,), jnp.float32),
            pltpu.SMEM((S,), jnp.float32),
        ],
        out_shape=out_shapes,
    )(superclass_scores, class_score, sub2super, target, weights)
    return (loss2d.reshape(()), fin, oh)


# speedup vs baseline: 2.3800x; 2.3800x over previous
import jax
import jax.numpy as jnp
from jax.experimental import pallas as pl
from jax.experimental.pallas import tpu as pltpu

_LAMBDA = 0.1
_SMOOTH = 1e-07


def _body(B, C, S, num_h):
    def body(sup_ref, cs_ref, s2s_ref, tgt_ref, w_ref,
             loss_ref, fin_ref, oh_ref,
             a_interc, a_sumc, a_cntc, a_sums, a_cnts, a_inters):
        b = pl.program_id(0)
        h = pl.program_id(1)

        @pl.when(jnp.logical_and(b == 0, h == 0))
        def _init():
            for c in range(C):
                a_interc[c] = 0.0
                a_sumc[c] = 0.0
                a_cntc[c] = 0.0
            for s in range(S):
                a_sums[s] = 0.0
                a_cnts[s] = 0.0
                a_inters[s] = 0.0

        t = tgt_ref[0]

        def class_body(c, st):
            oh = t == c
            ohf = oh.astype(jnp.float32)
            oh_ref[0, c] = ohf
            x = cs_ref[0, c]
            sidx = s2s_ref[c]
            g = sup_ref[0, sidx]
            fin_ref[0, c] = x * g
            a_interc[c] += jnp.sum(x * ohf)
            a_sumc[c] += jnp.sum(x)
            a_cntc[c] += jnp.sum(ohf)
            return st + sidx * oh.astype(jnp.int32)

        st = jax.lax.fori_loop(0, C, class_body, jnp.zeros_like(t))
        for s in range(S):
            sup_s = sup_ref[0, s]
            m = (st == s).astype(jnp.float32)
            a_sums[s] += jnp.sum(sup_s)
            a_cnts[s] += jnp.sum(m)
            a_inters[s] += jnp.sum(sup_s * m)

        @pl.when(jnp.logical_and(b == B - 1, h == num_h - 1))
        def _finish():
            sl = 0.0
            for s in range(S):
                sl += 1.0 - (2.0 * a_inters[s] + _SMOOTH) / (
                    a_sums[s] + a_cnts[s] + _SMOOTH)
            cl = 0.0
            wsum = 0.0
            for c in range(C):
                pc = 1.0 - (2.0 * a_interc[c] + _SMOOTH) / (
                    a_sumc[c] + a_cntc[c] + _SMOOTH)
                cl += pc * w_ref[c]
                wsum += w_ref[c]
            loss_ref[0, 0] = _LAMBDA * sl / S + cl / wsum

    return body


def kernel(superclass_scores, class_score, super2sub, target, weights):
    B, C, H, W = class_score.shape
    S = superclass_scores.shape[1]
    bh = 256
    num_h = H // bh

    cids = jnp.arange(C, dtype=jnp.int32)
    member = jnp.any(super2sub.astype(jnp.int32)[None, :, :] == cids[:, None, None],
                     axis=2)
    sub2super = jnp.sum(member.astype(jnp.int32)
                        * jnp.arange(S, dtype=jnp.int32)[None, :], axis=1)

    grid = (B, num_h)
    out_shapes = (
        jax.ShapeDtypeStruct((1, 1), jnp.float32),
        jax.ShapeDtypeStruct((B, C, H, W), jnp.float32),
        jax.ShapeDtypeStruct((B, C, H, W), jnp.float32),
    )
    loss2d, fin, oh = pl.pallas_call(
        _body(B, C, S, num_h),
        grid=grid,
        in_specs=[
            pl.BlockSpec((1, S, bh, W), lambda b, h: (b, 0, h, 0)),
            pl.BlockSpec((1, C, bh, W), lambda b, h: (b, 0, h, 0)),
            pl.BlockSpec(memory_space=pltpu.SMEM),
            pl.BlockSpec((1, bh, W), lambda b, h: (b, h, 0)),
            pl.BlockSpec(memory_space=pltpu.SMEM),
        ],
        out_specs=(
            pl.BlockSpec(memory_space=pltpu.SMEM),
            pl.BlockSpec((1, C, bh, W), lambda b, h: (b, 0, h, 0)),
            pl.BlockSpec((1, C, bh, W), lambda b, h: (b, 0, h, 0)),
        ),
        scratch_shapes=[
            pltpu.SMEM((C,), jnp.float32),
            pltpu.SMEM((C,), jnp.float32),
            pltpu.SMEM((C,), jnp.float32),
            pltpu.SMEM((S,), jnp.float32),
            pltpu.SMEM((S,), jnp.float32),
            pltpu.SMEM((S,), jnp.float32),
        ],
        out_shape=out_shapes,
    )(superclass_scores, class_score, sub2super, target, weights)
    return (loss2d.reshape(()), fin, oh)


# speedup vs baseline: 2.4058x; 1.0109x over previous
import jax
import jax.numpy as jnp
from jax.experimental import pallas as pl
from jax.experimental.pallas import tpu as pltpu

_LAMBDA = 0.1
_SMOOTH = 1e-07


def _body(B, C, S, num_h):
    def body(sup_ref, cs_ref, s2s_ref, tgt_ref, w_ref,
             loss_ref, fin_ref, oh_ref,
             a_interc, a_sumc, a_cntc, a_intsup, a_sums):
        b = pl.program_id(0)
        h = pl.program_id(1)

        @pl.when(jnp.logical_and(b == 0, h == 0))
        def _init():
            for c in range(C):
                a_interc[c] = 0.0
                a_sumc[c] = 0.0
                a_cntc[c] = 0.0
                a_intsup[c] = 0.0
            for s in range(S):
                a_sums[s] = 0.0

        t = tgt_ref[0]

        def class_body(c, carry):
            oh = t == c
            ohf = oh.astype(jnp.float32)
            oh_ref[0, c] = ohf
            x = cs_ref[0, c]
            sidx = s2s_ref[c]
            g = sup_ref[0, sidx]
            fin_ref[0, c] = x * g
            a_interc[c] += jnp.sum(x * ohf)
            a_sumc[c] += jnp.sum(x)
            a_cntc[c] += jnp.sum(ohf)
            a_intsup[c] += jnp.sum(g * ohf)
            return carry

        jax.lax.fori_loop(0, C, class_body, 0)
        for s in range(S):
            a_sums[s] += jnp.sum(sup_ref[0, s])

        @pl.when(jnp.logical_and(b == B - 1, h == num_h - 1))
        def _finish():
            sl = 0.0
            for s in range(S):
                cnt_s = 0.0
                int_s = 0.0
                for c in range(C):
                    pred = s2s_ref[c] == s
                    cnt_s += jnp.where(pred, a_cntc[c], 0.0)
                    int_s += jnp.where(pred, a_intsup[c], 0.0)
                sl += 1.0 - (2.0 * int_s + _SMOOTH) / (
                    a_sums[s] + cnt_s + _SMOOTH)
            cl = 0.0
            wsum = 0.0
            for c in range(C):
                pc = 1.0 - (2.0 * a_interc[c] + _SMOOTH) / (
                    a_sumc[c] + a_cntc[c] + _SMOOTH)
                cl += pc * w_ref[c]
                wsum += w_ref[c]
            loss_ref[0, 0] = _LAMBDA * sl / S + cl / wsum

    return body


def kernel(superclass_scores, class_score, super2sub, target, weights):
    B, C, H, W = class_score.shape
    S = superclass_scores.shape[1]
    bh = 256
    num_h = H // bh

    cids = jnp.arange(C, dtype=jnp.int32)
    member = jnp.any(super2sub.astype(jnp.int32)[None, :, :] == cids[:, None, None],
                     axis=2)
    sub2super = jnp.sum(member.astype(jnp.int32)
                        * jnp.arange(S, dtype=jnp.int32)[None, :], axis=1)

    grid = (B, num_h)
    out_shapes = (
        jax.ShapeDtypeStruct((1, 1), jnp.float32),
        jax.ShapeDtypeStruct((B, C, H, W), jnp.float32),
        jax.ShapeDtypeStruct((B, C, H, W), jnp.float32),
    )
    loss2d, fin, oh = pl.pallas_call(
        _body(B, C, S, num_h),
        grid=grid,
        in_specs=[
            pl.BlockSpec((1, S, bh, W), lambda b, h: (b, 0, h, 0)),
            pl.BlockSpec((1, C, bh, W), lambda b, h: (b, 0, h, 0)),
            pl.BlockSpec(memory_space=pltpu.SMEM),
            pl.BlockSpec((1, bh, W), lambda b, h: (b, h, 0)),
            pl.BlockSpec(memory_space=pltpu.SMEM),
        ],
        out_specs=(
            pl.BlockSpec(memory_space=pltpu.SMEM),
            pl.BlockSpec((1, C, bh, W), lambda b, h: (b, 0, h, 0)),
            pl.BlockSpec((1, C, bh, W), lambda b, h: (b, 0, h, 0)),
        ),
        scratch_shapes=[
            pltpu.SMEM((C,), jnp.float32),
            pltpu.SMEM((C,), jnp.float32),
            pltpu.SMEM((C,), jnp.float32),
            pltpu.SMEM((C,), jnp.float32),
            pltpu.SMEM((S,), jnp.float32),
        ],
        out_shape=out_shapes,
    )(superclass_scores, class_score, sub2super, target, weights)
    return (loss2d.reshape(()), fin, oh)
